# Initial kernel scaffold; baseline (speedup 1.0000x reference)
import jax
import jax.numpy as jnp
from jax.experimental import pallas as pl


def kernel(inputs, embeddings):
    raise NotImplementedError("write your pallas kernel here")


# speedup vs baseline: 1.1087x; 1.1087x over previous
import functools

import jax
import jax.numpy as jnp
from jax import lax
from jax.experimental import pallas as pl
from jax.experimental.pallas import tpu as pltpu
from jax.experimental.pallas import tpu_sc as plsc

_NUM_E = 8192
_DIM = 256
_TT = 256
_W = 2048
_N_TILES = _NUM_E // _TT
_COMMIT = 0.25


def _dist_body(x_ref, emb_ref, xn_ref, en_ref, idx_ref, acc_ref):
    i = pl.program_id(0)

    @pl.when(i == 0)
    def _init():
        acc_ref[...] = jnp.zeros((1, 1), jnp.float32)

    prod = lax.dot_general(
        x_ref[...], emb_ref[...],
        dimension_numbers=(((1,), (1,)), ((), ())),
        preferred_element_type=jnp.float32,
    )
    d = (xn_ref[0, 0, :][:, None] - 2.0 * prod) + en_ref[0, :][None, :]
    cur_v = jnp.zeros((_TT,), jnp.float32)
    cur_i = jnp.zeros((_TT,), jnp.int32)
    dmin = jnp.zeros((_TT,), jnp.float32)
    for w in range(_NUM_E // _W):
        dw = d[:, w * _W:(w + 1) * _W]
        m_w = jnp.min(dw, axis=1)
        i_w = jnp.argmin(dw, axis=1).astype(jnp.int32) + w * _W
        m_w_bf = m_w.astype(jnp.bfloat16).astype(jnp.float32)
        if w == 0:
            cur_v, cur_i, dmin = m_w_bf, i_w, m_w
        else:
            better = m_w < cur_v
            cur_i = jnp.where(better, i_w, cur_i)
            cur_v = jnp.where(better, m_w_bf, cur_v)
            dmin = jnp.minimum(dmin, m_w)
    idx_ref[0, 0, :] = cur_i
    acc_ref[...] += jnp.sum(dmin).reshape(1, 1)


_dist_call = pl.pallas_call(
    _dist_body,
    grid=(_N_TILES,),
    in_specs=[
        pl.BlockSpec((_TT, _DIM), lambda i: (i, 0)),
        pl.BlockSpec((_NUM_E, _DIM), lambda i: (0, 0)),
        pl.BlockSpec((1, 1, _TT), lambda i: (i, 0, 0)),
        pl.BlockSpec((1, _NUM_E), lambda i: (0, 0)),
    ],
    out_specs=[
        pl.BlockSpec((1, 1, _TT), lambda i: (i, 0, 0)),
        pl.BlockSpec((1, 1), lambda i: (0, 0)),
    ],
    out_shape=[
        jax.ShapeDtypeStruct((_N_TILES, 1, _TT), jnp.int32),
        jax.ShapeDtypeStruct((1, 1), jnp.float32),
    ],
)


@functools.lru_cache(maxsize=None)
def _make_sc_gather():
    info = plsc.get_sparse_core_info()
    nw = info.num_cores * info.num_subcores
    bpw = _NUM_E // nw
    mesh = plsc.VectorSubcoreMesh(core_axis_name="c", subcore_axis_name="s")

    @functools.partial(
        pl.kernel, mesh=mesh,
        out_type=jax.ShapeDtypeStruct((_NUM_E, _DIM), jnp.float32),
        scratch_types=[
            pltpu.VMEM((bpw,), jnp.int32),
            pltpu.VMEM((bpw, _DIM), jnp.float32),
            pltpu.SemaphoreType.DMA,
        ],
    )
    def gather(table_hbm, idx_hbm, out_hbm, idx_v, rows_v, sem):
        wid = lax.axis_index("s") * info.num_cores + lax.axis_index("c")
        base = wid * bpw
        pltpu.sync_copy(idx_hbm.at[pl.ds(base, bpw)], idx_v)
        pltpu.async_copy(table_hbm.at[idx_v], rows_v, sem).wait()
        pltpu.sync_copy(rows_v, out_hbm.at[pl.ds(base, bpw)])

    return gather


def kernel(inputs, embeddings):
    x = inputs.reshape(-1, _DIM)
    xb = x.astype(jnp.bfloat16)
    eb = embeddings.astype(jnp.bfloat16)
    en = jnp.sum(embeddings ** 2, axis=1)
    xn = jnp.sum(x ** 2, axis=1)
    idx3, dsum = _dist_call(
        xb, eb,
        xn.reshape(_N_TILES, 1, _TT),
        en.reshape(1, _NUM_E),
    )
    idx_flat = idx3.reshape(-1)
    quantized = _make_sc_gather()(embeddings, idx_flat)
    quantized = quantized.reshape(inputs.shape)
    loss = (_COMMIT / x.size) * dsum[0, 0]
    enc = idx_flat.reshape(inputs.shape[0], -1)
    return quantized, loss, enc


# speedup vs baseline: 1.2337x; 1.1127x over previous
import functools

import jax
import jax.numpy as jnp
from jax import lax
from jax.experimental import pallas as pl
from jax.experimental.pallas import tpu as pltpu
from jax.experimental.pallas import tpu_sc as plsc

_NUM_E = 8192
_DIM = 256
_TT = 256
_W = 2048
_N_TILES = _NUM_E // _TT
_COMMIT = 0.25


def _dist_body(x_ref, emb_ref, xn_ref, en_ref, idx_ref, acc_ref):
    i = pl.program_id(0)

    @pl.when(i == 0)
    def _init():
        acc_ref[...] = jnp.zeros((1, 1), jnp.float32)

    prod = lax.dot_general(
        x_ref[...], emb_ref[...],
        dimension_numbers=(((1,), (1,)), ((), ())),
        preferred_element_type=jnp.float32,
    )
    d = (xn_ref[0, 0, :][:, None] + prod) + en_ref[0, :][None, :]
    cur_v = jnp.zeros((_TT,), jnp.float32)
    cur_i = jnp.zeros((_TT,), jnp.int32)
    dmin = jnp.zeros((_TT,), jnp.float32)
    for w in range(_NUM_E // _W):
        dw = d[:, w * _W:(w + 1) * _W]
        m_w = jnp.min(dw, axis=1)
        i_w = jnp.argmin(dw, axis=1).astype(jnp.int32) + w * _W
        m_w_bf = m_w.astype(jnp.bfloat16).astype(jnp.float32)
        if w == 0:
            cur_v, cur_i, dmin = m_w_bf, i_w, m_w
        else:
            better = m_w < cur_v
            cur_i = jnp.where(better, i_w, cur_i)
            cur_v = jnp.where(better, m_w_bf, cur_v)
            dmin = jnp.minimum(dmin, m_w)
    idx_ref[0, 0, :] = cur_i
    acc_ref[...] += jnp.sum(dmin).reshape(1, 1)


_dist_call = pl.pallas_call(
    _dist_body,
    grid=(_N_TILES,),
    in_specs=[
        pl.BlockSpec((_TT, _DIM), lambda i: (i, 0)),
        pl.BlockSpec((_NUM_E, _DIM), lambda i: (0, 0)),
        pl.BlockSpec((1, 1, _TT), lambda i: (i, 0, 0)),
        pl.BlockSpec((1, _NUM_E), lambda i: (0, 0)),
    ],
    out_specs=[
        pl.BlockSpec((1, 1, _TT), lambda i: (i, 0, 0)),
        pl.BlockSpec((1, 1), lambda i: (0, 0)),
    ],
    out_shape=[
        jax.ShapeDtypeStruct((_N_TILES, 1, _TT), jnp.int32),
        jax.ShapeDtypeStruct((1, 1), jnp.float32),
    ],
)


@functools.lru_cache(maxsize=None)
def _make_sc_gather():
    info = plsc.get_sparse_core_info()
    nw = info.num_cores * info.num_subcores
    bpw = _NUM_E // nw
    mesh = plsc.VectorSubcoreMesh(core_axis_name="c", subcore_axis_name="s")

    @functools.partial(
        pl.kernel, mesh=mesh,
        out_type=jax.ShapeDtypeStruct((_NUM_E, _DIM), jnp.float32),
        scratch_types=[
            pltpu.VMEM((bpw,), jnp.int32),
            pltpu.VMEM((bpw, _DIM), jnp.float32),
            pltpu.SemaphoreType.DMA,
        ],
    )
    def gather(table_hbm, idx_hbm, out_hbm, idx_v, rows_v, sem):
        wid = lax.axis_index("s") * info.num_cores + lax.axis_index("c")
        base = wid * bpw
        pltpu.sync_copy(idx_hbm.at[pl.ds(base, bpw)], idx_v)
        pltpu.async_copy(table_hbm.at[idx_v], rows_v, sem).wait()
        pltpu.sync_copy(rows_v, out_hbm.at[pl.ds(base, bpw)])

    return gather


def kernel(inputs, embeddings):
    x = inputs.reshape(-1, _DIM)
    xb = (-2.0 * x).astype(jnp.bfloat16)
    eb = embeddings.astype(jnp.bfloat16)
    en = jnp.sum(embeddings ** 2, axis=1)
    xn = jnp.sum(x ** 2, axis=1)
    idx3, dsum = _dist_call(
        xb, eb,
        xn.reshape(_N_TILES, 1, _TT),
        en.reshape(1, _NUM_E),
    )
    idx_flat = idx3.reshape(-1)
    quantized = _make_sc_gather()(embeddings, idx_flat)
    quantized = quantized.reshape(inputs.shape)
    loss = (_COMMIT / x.size) * dsum[0, 0]
    enc = idx_flat.reshape(inputs.shape[0], -1)
    return quantized, loss, enc
